# baseline (speedup 1.0000x reference)
import functools

import jax
import jax.numpy as jnp
from jax import lax
from jax.experimental import pallas as pl
from jax.experimental.pallas import tpu as pltpu
from jax.experimental.pallas import tpu_sc as plsc

N = 2097152
GX, GY, GZ = 512, 512, 128
LOWER_X, LOWER_Y, LOWER_Z = -4.0, -4.0, -1.0
RES = 64.0

NW = 32
SUBV = 128
LANES = 16


def _build(n_points, b_chunk):
  pw = n_points // NW
  b = min(b_chunk, pw)
  sub = b // SUBV
  nch = pw // b
  vps = SUBV // LANES
  assert pw % b == 0 and b % SUBV == 0 and nch % 2 == 0

  mesh = plsc.VectorSubcoreMesh(core_axis_name="c", subcore_axis_name="s")

  def one_set():
    return (
        [pltpu.VMEM((b,), jnp.float32) for _ in range(3)]
        + [pltpu.VMEM((b,), jnp.int32) for _ in range(4)]
        + [pltpu.VMEM((b,), jnp.int32) for _ in range(4)]
        + [pltpu.VMEM((b,), jnp.float32) for _ in range(4)]
        + [pltpu.VMEM((b,), jnp.float32)]
        + [pltpu.SemaphoreType.DMA]
    )

  @functools.partial(
      pl.kernel,
      out_type=jax.ShapeDtypeStruct((n_points,), jnp.float32),
      mesh=mesh,
      scratch_types=one_set() + one_set(),
      compiler_params=pltpu.CompilerParams(needs_layout_passes=False),
  )
  def vox(xs_hbm, ys_hbm, zs_hbm, g_hbm, out_hbm, *refs):
    sets = (refs[:17], refs[17:])
    wid = lax.axis_index("s") * 2 + lax.axis_index("c")
    base0 = wid * pw

    def load_comp_fire(ci, st):
      (px, py, pz,
       i00, i10, i01, i11,
       _d0, _d1, _d2, _d3,
       wfx, wfy, wfz, wvm, _ob, sem) = st
      idx = (i00, i10, i01, i11)
      base = base0 + ci * b
      pltpu.sync_copy(xs_hbm.at[pl.ds(base, b)], px)
      pltpu.sync_copy(ys_hbm.at[pl.ds(base, b)], py)
      pltpu.sync_copy(zs_hbm.at[pl.ds(base, b)], pz)

      def comp(j, c2):
        for t in range(vps):
          s = pl.ds(j * SUBV + t * LANES, LANES)
          gx = (px[s] - LOWER_X) * RES
          gy = (py[s] - LOWER_Y) * RES
          gz = (pz[s] - LOWER_Z) * RES
          i0x = jnp.clip(gx.astype(jnp.int32), 0, GX - 1)
          i0y = jnp.clip(gy.astype(jnp.int32), 0, GY - 1)
          i0z = jnp.clip(gz.astype(jnp.int32), 0, GZ - 1)
          valid = ((gx >= 0.0) & (gx <= GX - 1.0)
                   & (gy >= 0.0) & (gy <= GY - 1.0)
                   & (gz >= 0.0) & (gz <= GZ - 1.0))
          lx0 = i0x * (GY * GZ)
          lx1 = jnp.minimum(i0x + 1, GX - 1) * (GY * GZ)
          ly0 = i0y * GZ
          ly1 = jnp.minimum(i0y + 1, GY - 1) * GZ
          i00[s] = lx0 + ly0 + i0z
          i10[s] = lx1 + ly0 + i0z
          i01[s] = lx0 + ly1 + i0z
          i11[s] = lx1 + ly1 + i0z
          wfx[s] = gx - i0x.astype(jnp.float32)
          wfy[s] = gy - i0y.astype(jnp.float32)
          wfz[s] = gz - i0z.astype(jnp.float32)
          wvm[s] = jnp.where(valid, 1.0, 0.0)
        sj = pl.ds(j * SUBV, SUBV)
        for c in range(4):
          pltpu.async_copy(g_hbm.at[idx[c].at[sj]], st[7 + c].at[sj], sem)
        return c2

      lax.fori_loop(0, sub, comp, 0)

    def drain_interp_store(ci, st):
      (_px, _py, _pz,
       i00, i10, i01, i11,
       d00, d10, d01, d11,
       wfx, wfy, wfz, wvm, ob, sem) = st
      idx = (i00, i10, i01, i11)
      base = base0 + ci * b

      def interp(j, c2):
        sj = pl.ds(j * SUBV, SUBV)
        for c in range(4):
          pltpu.make_async_copy(g_hbm.at[idx[c].at[sj]],
                                st[7 + c].at[sj], sem).wait()
        for t in range(vps):
          s = pl.ds(j * SUBV + t * LANES, LANES)
          fx = wfx[s]
          fy = wfy[s]
          fz = wfz[s]
          vm = wvm[s]
          w00 = d00[s]
          w10 = d10[s]
          w01 = d01[s]
          w11 = d11[s]
          hi = jnp.int32(-65536)
          a00 = plsc.bitcast(w00 << 16, jnp.float32)
          b00 = plsc.bitcast(w00 & hi, jnp.float32)
          a10 = plsc.bitcast(w10 << 16, jnp.float32)
          b10 = plsc.bitcast(w10 & hi, jnp.float32)
          a01 = plsc.bitcast(w01 << 16, jnp.float32)
          b01 = plsc.bitcast(w01 & hi, jnp.float32)
          a11 = plsc.bitcast(w11 << 16, jnp.float32)
          b11 = plsc.bitcast(w11 & hi, jnp.float32)
          cz00 = a00 + fz * (b00 - a00)
          cz10 = a10 + fz * (b10 - a10)
          cz01 = a01 + fz * (b01 - a01)
          cz11 = a11 + fz * (b11 - a11)
          cx0 = cz00 + fx * (cz10 - cz00)
          cx1 = cz01 + fx * (cz11 - cz01)
          ob[s] = (cx0 + fy * (cx1 - cx0)) * vm
        return c2

      lax.fori_loop(0, sub, interp, 0)
      pltpu.sync_copy(ob, out_hbm.at[pl.ds(base, b)])

    load_comp_fire(0, sets[0])

    def pair(k, carry):
      ci = 2 * k
      load_comp_fire(ci + 1, sets[1])
      drain_interp_store(ci, sets[0])

      @pl.when(ci + 2 < nch)
      def _():
        load_comp_fire(ci + 2, sets[0])

      drain_interp_store(ci + 1, sets[1])
      return carry

    lax.fori_loop(0, nch // 2, pair, 0)

  return vox


_VOX = _build(N, 2048)


def kernel(x, grid):
  xs = x[:, 0]
  ys = x[:, 1]
  zs = x[:, 2]
  gb = grid.reshape(-1).astype(jnp.bfloat16)
  gn = jnp.concatenate([gb[1:], gb[-1:]])
  gp = jax.lax.bitcast_convert_type(
      jnp.stack([gb, gn], axis=-1), jnp.int32)
  sigma = _VOX(xs, ys, zs, gp)
  alpha = jnp.zeros((N,), jnp.float32)
  return sigma, alpha


# speedup vs baseline: 2.6437x; 2.6437x over previous
import functools

import jax
import jax.numpy as jnp
from jax import lax
from jax.experimental import pallas as pl
from jax.experimental.pallas import tpu as pltpu
from jax.experimental.pallas import tpu_sc as plsc

N = 2097152
GX, GY, GZ = 512, 512, 128
LOWER_X, LOWER_Y, LOWER_Z = -4.0, -4.0, -1.0
RES = 64.0

NW = 32
SUBV = 128
LANES = 16


def _build(n_points, b_chunk):
  pw = n_points // NW
  b = min(b_chunk, pw)
  sub = b // SUBV
  nch = pw // b
  vps = SUBV // LANES
  assert pw % b == 0 and b % SUBV == 0 and nch % 2 == 0

  mesh = plsc.VectorSubcoreMesh(core_axis_name="c", subcore_axis_name="s")

  def one_set():
    return (
        [pltpu.VMEM((b,), jnp.float32) for _ in range(3)]
        + [pltpu.VMEM((b,), jnp.int32) for _ in range(4)]
        + [pltpu.VMEM((b,), jnp.int32) for _ in range(4)]
        + [pltpu.VMEM((b,), jnp.float32) for _ in range(4)]
        + [pltpu.VMEM((b,), jnp.float32)]
        + [pltpu.SemaphoreType.DMA]
    )

  @functools.partial(
      pl.kernel,
      out_type=jax.ShapeDtypeStruct((n_points,), jnp.float32),
      mesh=mesh,
      scratch_types=one_set() + one_set(),
      compiler_params=pltpu.CompilerParams(needs_layout_passes=False),
  )
  def vox(xs_hbm, ys_hbm, zs_hbm, g_hbm, out_hbm, *refs):
    sets = (refs[:17], refs[17:])
    wid = lax.axis_index("s") * 2 + lax.axis_index("c")
    base0 = wid * pw

    def load_comp_fire(ci, st):
      (px, py, pz,
       i00, i10, i01, i11,
       _d0, _d1, _d2, _d3,
       wfx, wfy, wfz, wvm, _ob, sem) = st
      idx = (i00, i10, i01, i11)
      base = base0 + ci * b
      pltpu.sync_copy(xs_hbm.at[pl.ds(base, b)], px)
      pltpu.sync_copy(ys_hbm.at[pl.ds(base, b)], py)
      pltpu.sync_copy(zs_hbm.at[pl.ds(base, b)], pz)

      def comp(j, c2):
        for t in range(vps):
          s = pl.ds(j * SUBV + t * LANES, LANES)
          gx = (px[s] - LOWER_X) * RES
          gy = (py[s] - LOWER_Y) * RES
          gz = (pz[s] - LOWER_Z) * RES
          i0x = jnp.clip(gx.astype(jnp.int32), 0, GX - 1)
          i0y = jnp.clip(gy.astype(jnp.int32), 0, GY - 1)
          i0z = jnp.clip(gz.astype(jnp.int32), 0, GZ - 1)
          valid = ((gx >= 0.0) & (gx <= GX - 1.0)
                   & (gy >= 0.0) & (gy <= GY - 1.0)
                   & (gz >= 0.0) & (gz <= GZ - 1.0))
          lx0 = i0x * (GY * GZ)
          lx1 = jnp.minimum(i0x + 1, GX - 1) * (GY * GZ)
          ly0 = i0y * GZ
          ly1 = jnp.minimum(i0y + 1, GY - 1) * GZ
          i00[s] = lx0 + ly0 + i0z
          i10[s] = lx1 + ly0 + i0z
          i01[s] = lx0 + ly1 + i0z
          i11[s] = lx1 + ly1 + i0z
          wfx[s] = gx - i0x.astype(jnp.float32)
          wfy[s] = gy - i0y.astype(jnp.float32)
          wfz[s] = gz - i0z.astype(jnp.float32)
          wvm[s] = jnp.where(valid, 1.0, 0.0)
        sj = pl.ds(j * SUBV, SUBV)
        for c in range(4):
          pltpu.async_copy(g_hbm.at[idx[c].at[sj]], st[7 + c].at[sj], sem)
        return c2

      lax.fori_loop(0, sub, comp, 0)

    def drain_interp_store(ci, st):
      (_px, _py, _pz,
       i00, i10, i01, i11,
       d00, d10, d01, d11,
       wfx, wfy, wfz, wvm, ob, sem) = st
      idx = (i00, i10, i01, i11)
      base = base0 + ci * b

      def interp(j, c2):
        sj = pl.ds(j * SUBV, SUBV)
        for c in range(4):
          pltpu.make_async_copy(g_hbm.at[idx[c].at[sj]],
                                st[7 + c].at[sj], sem).wait()
        for t in range(vps):
          s = pl.ds(j * SUBV + t * LANES, LANES)
          fx = wfx[s]
          fy = wfy[s]
          fz = wfz[s]
          vm = wvm[s]
          w00 = d00[s]
          w10 = d10[s]
          w01 = d01[s]
          w11 = d11[s]
          hi = jnp.int32(-65536)
          a00 = plsc.bitcast(w00 << 16, jnp.float32)
          b00 = plsc.bitcast(w00 & hi, jnp.float32)
          a10 = plsc.bitcast(w10 << 16, jnp.float32)
          b10 = plsc.bitcast(w10 & hi, jnp.float32)
          a01 = plsc.bitcast(w01 << 16, jnp.float32)
          b01 = plsc.bitcast(w01 & hi, jnp.float32)
          a11 = plsc.bitcast(w11 << 16, jnp.float32)
          b11 = plsc.bitcast(w11 & hi, jnp.float32)
          cz00 = a00 + fz * (b00 - a00)
          cz10 = a10 + fz * (b10 - a10)
          cz01 = a01 + fz * (b01 - a01)
          cz11 = a11 + fz * (b11 - a11)
          cx0 = cz00 + fx * (cz10 - cz00)
          cx1 = cz01 + fx * (cz11 - cz01)
          ob[s] = (cx0 + fy * (cx1 - cx0)) * vm
        return c2

      lax.fori_loop(0, sub, interp, 0)
      pltpu.sync_copy(ob, out_hbm.at[pl.ds(base, b)])

    load_comp_fire(0, sets[0])

    def pair(k, carry):
      ci = 2 * k
      load_comp_fire(ci + 1, sets[1])
      drain_interp_store(ci, sets[0])

      @pl.when(ci + 2 < nch)
      def _():
        load_comp_fire(ci + 2, sets[0])

      drain_interp_store(ci + 1, sets[1])
      return carry

    lax.fori_loop(0, nch // 2, pair, 0)

  return vox


_VOX = _build(N, 2048)


def kernel(x, grid):
  xs = x[:, 0]
  ys = x[:, 1]
  zs = x[:, 2]
  gi = jax.lax.bitcast_convert_type(grid.reshape(-1), jnp.int32)
  gn = jnp.concatenate([gi[1:], gi[-1:]])
  gp = jax.lax.shift_right_logical(gi, 16) | (gn & jnp.int32(-65536))
  sigma = _VOX(xs, ys, zs, gp)
  alpha = jnp.zeros((N,), jnp.float32)
  return sigma, alpha
